# Initial kernel scaffold; baseline (speedup 1.0000x reference)
import jax
import jax.numpy as jnp
from jax.experimental import pallas as pl


def kernel(idx, wte, wpe):
    raise NotImplementedError("write your pallas kernel here")


# speedup vs baseline: 1.0712x; 1.0712x over previous
import functools

import jax
import jax.numpy as jnp
from jax import lax
from jax.experimental import pallas as pl
from jax.experimental.pallas import tpu as pltpu
from jax.experimental.pallas import tpu_sc as plsc

_NUM_CORES = 2
_NUM_SUBCORES = 16
_NUM_WORKERS = _NUM_CORES * _NUM_SUBCORES
_CHUNK = 64
_LANES = 16


def _emb_lookup(idx_flat, wte, wpe):
    n = idx_flat.shape[0]
    _, d = wte.shape
    s = wpe.shape[0]
    per_w = n // _NUM_WORKERS
    n_chunks = per_w // _CHUNK
    slices_per_row = d // _LANES
    mesh = plsc.VectorSubcoreMesh(core_axis_name="c", subcore_axis_name="s")

    @functools.partial(
        pl.kernel,
        out_type=jax.ShapeDtypeStruct((n, d), jnp.float32),
        mesh=mesh,
        scratch_types=[
            pltpu.VMEM((per_w,), jnp.int32),
            pltpu.VMEM((_CHUNK, d), jnp.float32),
            pltpu.VMEM((_CHUNK, d), jnp.float32),
            pltpu.SemaphoreType.DMA,
        ],
    )
    def body(idx_hbm, wte_hbm, wpe_hbm, out_hbm, idx_v, buf_w, buf_p, sem):
        wid = lax.axis_index("s") * _NUM_CORES + lax.axis_index("c")
        base = wid * per_w
        pltpu.sync_copy(idx_hbm.at[pl.ds(base, per_w)], idx_v)
        s_base = lax.rem(base, s)

        def chunk(i, carry):
            off = i * _CHUNK
            gather = pltpu.async_copy(
                wte_hbm.at[idx_v.at[pl.ds(off, _CHUNK)]], buf_w, sem
            )
            pltpu.sync_copy(wpe_hbm.at[pl.ds(s_base + off, _CHUNK)], buf_p)
            gather.wait()

            def add_row(r, carry2):
                for j in range(slices_per_row):
                    x = buf_p[r, pl.ds(j * _LANES, _LANES)]
                    plsc.addupdate(buf_w.at[r, pl.ds(j * _LANES, _LANES)], x)
                return carry2

            lax.fori_loop(0, _CHUNK, add_row, 0)
            pltpu.sync_copy(buf_w, out_hbm.at[pl.ds(base + off, _CHUNK)])
            return carry

        lax.fori_loop(0, n_chunks, chunk, 0)

    return body(idx_flat, wte, wpe)


def kernel(idx, wte, wpe):
    b, s = idx.shape
    d = wte.shape[1]
    idx_flat = idx.reshape(b * s).astype(jnp.int32)
    out = _emb_lookup(idx_flat, wte, wpe)
    return out.reshape(b, s, d)


# speedup vs baseline: 1.2002x; 1.1203x over previous
import functools

import jax
import jax.numpy as jnp
from jax import lax
from jax.experimental import pallas as pl
from jax.experimental.pallas import tpu as pltpu
from jax.experimental.pallas import tpu_sc as plsc

_NUM_CORES = 2
_NUM_SUBCORES = 16
_NUM_WORKERS = _NUM_CORES * _NUM_SUBCORES
_CHUNK = 32
_NBUF_W = 3
_NBUF_P = 2
_LANES = 16


def _emb_lookup(idx_flat, wte, wpe):
    n = idx_flat.shape[0]
    _, d = wte.shape
    s = wpe.shape[0]
    per_w = n // _NUM_WORKERS
    n_chunks = per_w // _CHUNK
    slices_per_row = d // _LANES
    mesh = plsc.VectorSubcoreMesh(core_axis_name="c", subcore_axis_name="s")

    scratch = (
        [pltpu.VMEM((per_w,), jnp.int32)]
        + [pltpu.VMEM((_CHUNK, d), jnp.float32)] * (_NBUF_W + _NBUF_P)
        + [pltpu.SemaphoreType.DMA] * (2 * _NBUF_W + _NBUF_P)
    )

    @functools.partial(
        pl.kernel,
        out_type=jax.ShapeDtypeStruct((n, d), jnp.float32),
        mesh=mesh,
        scratch_types=scratch,
    )
    def body(idx_hbm, wte_hbm, wpe_hbm, out_hbm, idx_v, *rest):
        bufw = rest[:_NBUF_W]
        bufp = rest[_NBUF_W:_NBUF_W + _NBUF_P]
        gsem = rest[_NBUF_W + _NBUF_P:2 * _NBUF_W + _NBUF_P]
        psem = rest[2 * _NBUF_W + _NBUF_P:2 * _NBUF_W + 2 * _NBUF_P]
        osem = rest[2 * _NBUF_W + 2 * _NBUF_P:]

        wid = lax.axis_index("s") * _NUM_CORES + lax.axis_index("c")
        base = wid * per_w
        pltpu.sync_copy(idx_hbm.at[pl.ds(base, per_w)], idx_v)
        s_base = lax.rem(base, s)

        def issue(i):
            off = i * _CHUNK
            g = pltpu.async_copy(
                wte_hbm.at[idx_v.at[pl.ds(off, _CHUNK)]],
                bufw[i % _NBUF_W],
                gsem[i % _NBUF_W],
            )
            p = pltpu.async_copy(
                wpe_hbm.at[pl.ds(s_base + off, _CHUNK)],
                bufp[i % _NBUF_P],
                psem[i % _NBUF_P],
            )
            return g, p

        inflight = {i: issue(i) for i in range(min(2, n_chunks))}
        out_h = {}
        for i in range(n_chunks):
            g, p = inflight.pop(i)
            g.wait()
            p.wait()
            pbuf = bufp[i % _NBUF_P]
            wbuf = bufw[i % _NBUF_W]

            def add_row(r, carry):
                for j in range(slices_per_row):
                    x = pbuf[r, pl.ds(j * _LANES, _LANES)]
                    plsc.addupdate(wbuf.at[r, pl.ds(j * _LANES, _LANES)], x)
                return carry

            lax.fori_loop(0, _CHUNK, add_row, 0)
            out_h[i] = pltpu.async_copy(
                wbuf,
                out_hbm.at[pl.ds(base + i * _CHUNK, _CHUNK)],
                osem[i % _NBUF_W],
            )
            if i + 2 < n_chunks:
                if i - 1 in out_h:
                    out_h.pop(i - 1).wait()
                inflight[i + 2] = issue(i + 2)
        for i in sorted(out_h):
            out_h.pop(i).wait()

    return body(idx_flat, wte, wpe)


def kernel(idx, wte, wpe):
    b, s = idx.shape
    d = wte.shape[1]
    idx_flat = idx.reshape(b * s).astype(jnp.int32)
    out = _emb_lookup(idx_flat, wte, wpe)
    return out.reshape(b, s, d)
